# baseline (speedup 1.0000x reference)
import functools

import jax
import jax.numpy as jnp
from jax import lax
from jax.experimental import pallas as pl
from jax.experimental.pallas import tpu as pltpu
from jax.experimental.pallas import tpu_sc as plsc

N = 10000
E = 320000
D = 128
NC = 2
NS = 16
NW = NC * NS
EPT = E // NW
CH = 128
CPB = 80
EPAD = NW * CPB * CH
NGB = 2
NI = 4
UN = 4
RPT = N // NS
ZC = 125
DCH = 2000
BN = 1000
GRID = N // BN

_sc_mesh = plsc.VectorSubcoreMesh(core_axis_name="c", subcore_axis_name="s")


@functools.partial(
    pl.kernel,
    out_type=jax.ShapeDtypeStruct((NW, N), jnp.float32),
    mesh=_sc_mesh,
    compiler_params=pltpu.CompilerParams(needs_layout_passes=False),
    scratch_types=[
        pltpu.VMEM((N,), jnp.float32),
        pltpu.VMEM((DCH,), jnp.int32),
        pltpu.VMEM((DCH,), jnp.float32),
    ],
)
def _deg_kernel(col_hbm, w_hbm, out_hbm, acc, colbuf, wbuf):
    cid = lax.axis_index("c")
    sid = lax.axis_index("s")
    wid = cid * NS + sid

    def zero_body(i, _):
        acc[pl.ds(i * 16, 16)] = jnp.zeros((16,), jnp.float32)
        return 0

    lax.fori_loop(0, N // 16, zero_body, 0)

    def chunk_body(i, _):
        base = wid * EPT + i * DCH
        pltpu.sync_copy(col_hbm.at[pl.ds(base, DCH)], colbuf)
        pltpu.sync_copy(w_hbm.at[pl.ds(base, DCH)], wbuf)

        def grp(g, _):
            idx = colbuf[pl.ds(g * 16, 16)]
            val = wbuf[pl.ds(g * 16, 16)]
            plsc.addupdate_scatter(acc, [idx], val)
            return 0

        lax.fori_loop(0, DCH // 16, grp, 0)
        return 0

    lax.fori_loop(0, EPT // DCH, chunk_body, 0)
    pltpu.sync_copy(acc, out_hbm.at[wid])


@functools.partial(
    pl.kernel,
    out_type=jax.ShapeDtypeStruct((NC, N, D), jnp.float32),
    mesh=_sc_mesh,
    compiler_params=pltpu.CompilerParams(needs_layout_passes=False,
                                         use_tc_tiling_on_sc=False),
    scratch_types=[
        pltpu.VMEM_SHARED((N, D), jnp.float32),
        pltpu.VMEM((NI, 2, CH), jnp.int32),
        pltpu.VMEM((NI, CH), jnp.float32),
        pltpu.VMEM((NGB * CH, D), jnp.float32),
        pltpu.VMEM((CH, D), jnp.float32),
        [pltpu.SemaphoreType.DMA] * NI,
        [pltpu.SemaphoreType.DMA] * NGB,
        pltpu.SemaphoreType.DMA,
    ],
)
def _agg_kernel(hp_hbm, rc_hbm, w_hbm, out_hbm,
                acc_sh, idxring, wring, gbuf, sbuf, isems, gsems, ssem):
    cid = lax.axis_index("c")
    sid = lax.axis_index("s")
    wid = cid * NS + sid

    def zb(r, _):
        for f in range(D // 16):
            sbuf[r, pl.ds(f * 16, 16)] = jnp.zeros((16,), jnp.float32)
        return 0

    lax.fori_loop(0, ZC, zb, 0)
    for k in range(RPT // ZC):
        pltpu.sync_copy(sbuf.at[pl.ds(0, ZC)],
                        acc_sh.at[pl.ds(sid * RPT + k * ZC, ZC)])
    plsc.subcore_barrier()

    def issue_idx(j, m):
        pltpu.async_copy(rc_hbm.at[wid * CPB + j], idxring.at[m], isems[m])
        pltpu.async_copy(w_hbm.at[wid * CPB + j], wring.at[m], isems[m])

    def wait_idx(m):
        pltpu.make_async_copy(rc_hbm.at[wid * CPB], idxring.at[m],
                              isems[m]).wait()
        pltpu.make_async_copy(w_hbm.at[wid * CPB], wring.at[m],
                              isems[m]).wait()

    def gslice(b):
        return gbuf.at[pl.ds(b * CH, CH)]

    def issue_gather(m, b):
        pltpu.async_copy(hp_hbm.at[idxring.at[m, 0]], gslice(b), gsems[b])

    def wait_gather(m, b):
        pltpu.make_async_copy(hp_hbm.at[idxring.at[m, 0]], gslice(b),
                              gsems[b]).wait()

    def issue_scatter(m):
        pltpu.async_copy(sbuf, acc_sh.at[idxring.at[m, 1]], ssem, add=True)

    def wait_scatter(m):
        pltpu.make_async_copy(sbuf, acc_sh.at[idxring.at[m, 1]], ssem).wait()

    def scale_chunk(b, m):
        def make_grp(h):
            def grp(g, _):
                w16 = wring[m, pl.ds(g * 16, 16)]
                for e in range(h, h + 8):
                    w_s = w16[e]
                    r = b * CH + g * 16 + e
                    o = g * 16 + e
                    for f in range(D // 16):
                        sbuf[o, pl.ds(f * 16, 16)] = (
                            gbuf[r, pl.ds(f * 16, 16)] * w_s)
                return 0
            return grp

        lax.fori_loop(0, CH // 16, make_grp(0), 0)
        lax.fori_loop(0, CH // 16, make_grp(8), 0)

    for k in range(3):
        issue_idx(k, k)
    wait_idx(0)
    issue_gather(0, 0)
    wait_idx(1)
    issue_gather(1, 1)

    T = CPB // UN

    def block(t, _):
        for u in range(UN):
            b = u % NGB
            m = u
            mg = (u + 2) % NI
            mi = (u + 3) % NI
            j = t * UN + u

            wait_gather(m, b)
            if u == 0:
                @pl.when(t >= 1)
                def _():
                    wait_scatter(m)
                issue_idx(j + 3, mi)
            else:
                wait_scatter(m)

                @pl.when(t < T - 1)
                def _():
                    issue_idx(j + 3, mi)
            scale_chunk(b, m)
            if u < 2:
                wait_idx(mg)
                issue_gather(mg, b)
            else:
                @pl.when(t < T - 1)
                def _():
                    wait_idx(mg)
                    issue_gather(mg, b)
            issue_scatter(m)
        return 0

    lax.fori_loop(0, T, block, 0)
    wait_scatter((CPB - 1) % NI)

    plsc.subcore_barrier()

    def copy_out(k, _):
        base = sid * RPT + k * ZC
        pltpu.sync_copy(acc_sh.at[pl.ds(base, ZC)],
                        out_hbm.at[cid, pl.ds(base, ZC)])
        return 0

    lax.fori_loop(0, RPT // ZC, copy_out, 0)


def _dis_from(degp):
    deg = jnp.sum(degp, axis=-1) + 1.0
    return jnp.where(deg > 0, lax.rsqrt(deg), 0.0)


def _ln_relu(y, g, be):
    mu = jnp.mean(y, axis=-1, keepdims=True)
    var = jnp.mean((y - mu) ** 2, axis=-1, keepdims=True)
    return jnp.maximum((y - mu) * lax.rsqrt(var + 1e-5) * g + be, 0.0)


def _tc_prep_body(degp_ref, x_ref, w0_ref, hp_ref):
    dis = _dis_from(degp_ref[...])
    hp_ref[...] = jnp.dot(x_ref[...], w0_ref[...],
                          preferred_element_type=jnp.float32) * dis[:, None]


def _tc_post_body(degp_ref, x_ref, hp_ref, acc_ref, b_ref, g_ref, be_ref,
                  wn_ref, wjk_ref, jk_ref, xn_ref, hpn_ref, jko_ref,
                  *, first):
    dis = _dis_from(degp_ref[...])
    o = dis[:, None] * (acc_ref[0] + acc_ref[1] + hp_ref[...]) + b_ref[...]
    xn = _ln_relu(x_ref[...] + o, g_ref[...], be_ref[...])
    xn_ref[...] = xn
    hpn_ref[...] = jnp.dot(xn, wn_ref[...],
                           preferred_element_type=jnp.float32) * dis[:, None]
    jk = jnp.dot(xn, wjk_ref[...], preferred_element_type=jnp.float32)
    if not first:
        jk = jk + jk_ref[...]
    jko_ref[...] = jk


def _tc_final_body(degp_ref, x_ref, hp_ref, acc_ref, b_ref, g_ref, be_ref,
                   wjk_ref, bjk_ref, jk_ref, out_ref):
    dis = _dis_from(degp_ref[...])
    o = dis[:, None] * (acc_ref[0] + acc_ref[1] + hp_ref[...]) + b_ref[...]
    xn = _ln_relu(x_ref[...] + o, g_ref[...], be_ref[...])
    out_ref[...] = (jk_ref[...] + bjk_ref[...]
                    + jnp.dot(xn, wjk_ref[...],
                              preferred_element_type=jnp.float32))


_b_degp = pl.BlockSpec((BN, NW), lambda i: (i, 0))
_b_rows = pl.BlockSpec((BN, D), lambda i: (i, 0))
_b_acc = pl.BlockSpec((NC, BN, D), lambda i: (0, i, 0))
_b_w = pl.BlockSpec((D, D), lambda i: (0, 0))
_b_vec = pl.BlockSpec((1, D), lambda i: (0, 0))

_f32 = jnp.float32
_nd = jax.ShapeDtypeStruct((N, D), _f32)

_tc_prep = pl.pallas_call(
    _tc_prep_body,
    grid=(GRID,),
    in_specs=[_b_degp, _b_rows, _b_w],
    out_specs=_b_rows,
    out_shape=_nd,
)


def _make_post(first):
    return pl.pallas_call(
        functools.partial(_tc_post_body, first=first),
        grid=(GRID,),
        in_specs=[_b_degp, _b_rows, _b_rows, _b_acc, _b_vec, _b_vec, _b_vec,
                  _b_w, _b_w, _b_rows],
        out_specs=[_b_rows, _b_rows, _b_rows],
        out_shape=[_nd, _nd, _nd],
    )


_tc_post0 = _make_post(True)
_tc_post1 = _make_post(False)

_tc_final = pl.pallas_call(
    _tc_final_body,
    grid=(GRID,),
    in_specs=[_b_degp, _b_rows, _b_rows, _b_acc, _b_vec, _b_vec, _b_vec,
              _b_w, _b_vec, _b_rows],
    out_specs=_b_rows,
    out_shape=_nd,
)


def kernel(node, edge_index, edge_attr, batch_ptr,
           W0, b0, g0, be0, W1, b1, g1, be1, W2, b2, g2, be2,
           Wjk, bjk):
    del batch_ptr
    row = edge_index[0].astype(jnp.int32)
    col = edge_index[1].astype(jnp.int32)
    w = edge_attr.astype(jnp.float32)

    degp = _deg_kernel(col, w).T

    pad = EPAD - E
    rowp = jnp.concatenate([row, jnp.zeros((pad,), jnp.int32)])
    colp = jnp.concatenate([col, jnp.zeros((pad,), jnp.int32)])
    wp = jnp.concatenate([w, jnp.zeros((pad,), jnp.float32)])
    rc = jnp.stack([rowp.reshape(NW * CPB, CH),
                    colp.reshape(NW * CPB, CH)], axis=1)
    wchunk = wp.reshape(NW * CPB, CH)

    b0r, g0r, be0r = b0.reshape(1, D), g0.reshape(1, D), be0.reshape(1, D)
    b1r, g1r, be1r = b1.reshape(1, D), g1.reshape(1, D), be1.reshape(1, D)
    b2r, g2r, be2r = b2.reshape(1, D), g2.reshape(1, D), be2.reshape(1, D)
    wjk0, wjk1, wjk2 = Wjk[:D], Wjk[D:2 * D], Wjk[2 * D:]
    bjkr = bjk.reshape(1, D)

    hp0 = _tc_prep(degp, node, W0)
    acc0 = _agg_kernel(hp0, rc, wchunk)
    x1, hp1, jk = _tc_post0(degp, node, hp0, acc0, b0r, g0r, be0r,
                            W1, wjk0, jnp.zeros((N, D), _f32))
    acc1 = _agg_kernel(hp1, rc, wchunk)
    x2, hp2, jk = _tc_post1(degp, x1, hp1, acc1, b1r, g1r, be1r,
                            W2, wjk1, jk)
    acc2 = _agg_kernel(hp2, rc, wchunk)
    return _tc_final(degp, x2, hp2, acc2, b2r, g2r, be2r, wjk2, bjkr, jk)


# speedup vs baseline: 1.7162x; 1.7162x over previous
import functools

import jax
import jax.numpy as jnp
import numpy as np
from jax import lax
from jax.experimental import pallas as pl
from jax.experimental.pallas import tpu as pltpu
from jax.experimental.pallas import tpu_sc as plsc

N = 10000
E = 320000
D = 128
NC = 2
NS = 16
NW = NC * NS
EPT = E // NW
CH = 80
CPB = 128
EPAD = NW * CPB * CH
NB = 4
NSB = 2
NI = 8
UN = 8
RPT = N // NS
ZC = 125
DCH = 2000
BN = 1000
GRID = N // BN

_H = np.empty((D,), dtype=np.int32)
for _k in range(4):
    for _i in range(16):
        for _p in range(2):
            _H[32 * _k + 2 * _i + _p] = 32 * _k + 16 * _p + _i

_sc_mesh = plsc.VectorSubcoreMesh(core_axis_name="c", subcore_axis_name="s")


@functools.partial(
    pl.kernel,
    out_type=jax.ShapeDtypeStruct((NW, N), jnp.float32),
    mesh=_sc_mesh,
    compiler_params=pltpu.CompilerParams(needs_layout_passes=False),
    scratch_types=[
        pltpu.VMEM((N,), jnp.float32),
        pltpu.VMEM((DCH,), jnp.int32),
        pltpu.VMEM((DCH,), jnp.float32),
    ],
)
def _deg_kernel(col_hbm, w_hbm, out_hbm, acc, colbuf, wbuf):
    cid = lax.axis_index("c")
    sid = lax.axis_index("s")
    wid = cid * NS + sid

    def zero_body(i, _):
        acc[pl.ds(i * 16, 16)] = jnp.zeros((16,), jnp.float32)
        return 0

    lax.fori_loop(0, N // 16, zero_body, 0)

    def chunk_body(i, _):
        base = wid * EPT + i * DCH
        pltpu.sync_copy(col_hbm.at[pl.ds(base, DCH)], colbuf)
        pltpu.sync_copy(w_hbm.at[pl.ds(base, DCH)], wbuf)

        def grp(g, _):
            idx = colbuf[pl.ds(g * 16, 16)]
            val = wbuf[pl.ds(g * 16, 16)]
            plsc.addupdate_scatter(acc, [idx], val)
            return 0

        lax.fori_loop(0, DCH // 16, grp, 0)
        return 0

    lax.fori_loop(0, EPT // DCH, chunk_body, 0)
    pltpu.sync_copy(acc, out_hbm.at[wid])


@functools.partial(
    pl.kernel,
    out_type=jax.ShapeDtypeStruct((NC, N, D), jnp.float32),
    mesh=_sc_mesh,
    compiler_params=pltpu.CompilerParams(needs_layout_passes=False,
                                         use_tc_tiling_on_sc=False),
    scratch_types=[
        pltpu.VMEM_SHARED((N, D), jnp.float32),
        pltpu.VMEM((NI, 2, CH), jnp.int32),
        pltpu.VMEM((NI, CH), jnp.float32),
        pltpu.VMEM((NB * CH, D), jnp.bfloat16),
        pltpu.VMEM((NSB * CH, D), jnp.float32),
        [pltpu.SemaphoreType.DMA] * NI,
        [pltpu.SemaphoreType.DMA] * NB,
        [pltpu.SemaphoreType.DMA] * NSB,
    ],
)
def _agg_kernel(hpb_hbm, rc_hbm, w_hbm, out_hbm,
                acc_sh, idxring, wring, gbuf, sbuf, isems, gsems, ssems):
    cid = lax.axis_index("c")
    sid = lax.axis_index("s")
    wid = cid * NS + sid

    def zb(r, _):
        for f in range(D // 16):
            sbuf[r, pl.ds(f * 16, 16)] = jnp.zeros((16,), jnp.float32)
        return 0

    lax.fori_loop(0, ZC, zb, 0)
    for k in range(RPT // ZC):
        pltpu.sync_copy(sbuf.at[pl.ds(0, ZC)],
                        acc_sh.at[pl.ds(sid * RPT + k * ZC, ZC)])
    plsc.subcore_barrier()

    def issue_idx(j, m):
        pltpu.async_copy(rc_hbm.at[wid * CPB + j], idxring.at[m], isems[m])
        pltpu.async_copy(w_hbm.at[wid * CPB + j], wring.at[m], isems[m])

    def wait_idx(m):
        pltpu.make_async_copy(rc_hbm.at[wid * CPB], idxring.at[m],
                              isems[m]).wait()
        pltpu.make_async_copy(w_hbm.at[wid * CPB], wring.at[m],
                              isems[m]).wait()

    def gslice(b):
        return gbuf.at[pl.ds(b * CH, CH)]

    def sslice(s):
        return sbuf.at[pl.ds(s * CH, CH)]

    def issue_gather(m, b):
        pltpu.async_copy(hpb_hbm.at[idxring.at[m, 0]], gslice(b), gsems[b])

    def wait_gather(m, b):
        pltpu.make_async_copy(hpb_hbm.at[idxring.at[m, 0]], gslice(b),
                              gsems[b]).wait()

    def issue_scatter(m, s):
        pltpu.async_copy(sslice(s), acc_sh.at[idxring.at[m, 1]], ssems[s],
                         add=True)

    def wait_scatter(m, s):
        pltpu.make_async_copy(sslice(s), acc_sh.at[idxring.at[m, 1]],
                              ssems[s]).wait()

    def scale_chunk(b, s, m):
        def make_grp(h):
            def grp(g, _):
                w16 = wring[m, pl.ds(g * 16, 16)]
                for e in range(h, h + 8):
                    w_s = w16[e]
                    r = b * CH + g * 16 + e
                    o = s * CH + g * 16 + e
                    for c in range(D // 32):
                        v = gbuf[r, pl.ds(c * 32, 32)]
                        lo, hi = plsc.unpack(
                            v, format=plsc.PackFormat.INTERLEAVED)
                        sbuf[o, pl.ds(c * 32, 16)] = lo * w_s
                        sbuf[o, pl.ds(c * 32 + 16, 16)] = hi * w_s
                return 0
            return grp

        lax.fori_loop(0, CH // 16, make_grp(0), 0)
        lax.fori_loop(0, CH // 16, make_grp(8), 0)

    for k in range(6):
        issue_idx(k, k)
    for k in range(4):
        wait_idx(k)
        issue_gather(k, k)

    T = CPB // UN

    def block(t, _):
        for u in range(UN):
            j = t * UN + u
            b = u % NB
            s = u % NSB
            m = u
            m4 = (u + 4) % NI
            m6 = (u + 6) % NI

            wait_gather(m, b)
            if u < 2:
                @pl.when(t >= 1)
                def _():
                    wait_scatter(m, s)
            else:
                wait_scatter(m, s)
            if u < 2:
                issue_idx(j + 6, m6)
            else:
                @pl.when(t < T - 1)
                def _():
                    issue_idx(j + 6, m6)
            scale_chunk(b, s, m)
            if u < 4:
                wait_idx(m4)
                issue_gather(m4, b)
            else:
                @pl.when(t < T - 1)
                def _():
                    wait_idx(m4)
                    issue_gather(m4, b)
            issue_scatter(m, s)
        return 0

    lax.fori_loop(0, T, block, 0)
    wait_scatter((CPB - 2) % NI, 0)
    wait_scatter((CPB - 1) % NI, 1)

    plsc.subcore_barrier()

    def copy_out(k, _):
        base = sid * RPT + k * ZC
        pltpu.sync_copy(acc_sh.at[pl.ds(base, ZC)],
                        out_hbm.at[cid, pl.ds(base, ZC)])
        return 0

    lax.fori_loop(0, RPT // ZC, copy_out, 0)


def _dis_from(degp):
    deg = jnp.sum(degp, axis=-1) + 1.0
    return jnp.where(deg > 0, lax.rsqrt(deg), 0.0)


def _ln_relu(y, g, be):
    mu = jnp.mean(y, axis=-1, keepdims=True)
    var = jnp.mean((y - mu) ** 2, axis=-1, keepdims=True)
    return jnp.maximum((y - mu) * lax.rsqrt(var + 1e-5) * g + be, 0.0)


def _tc_prep_body(degp_ref, x_ref, w0_ref, wh0_ref, hp_ref, hpb_ref):
    dis = _dis_from(degp_ref[...])
    x = x_ref[...]
    hp_ref[...] = jnp.dot(x, w0_ref[...],
                          preferred_element_type=jnp.float32) * dis[:, None]
    hpb_ref[...] = (jnp.dot(x, wh0_ref[...], preferred_element_type=jnp.float32)
                    * dis[:, None]).astype(jnp.bfloat16)


def _tc_post_body(degp_ref, x_ref, hp_ref, acc_ref, b_ref, g_ref, be_ref,
                  wn_ref, whn_ref, wjk_ref, jk_ref,
                  xn_ref, hpn_ref, hpnb_ref, jko_ref, *, first):
    dis = _dis_from(degp_ref[...])
    o = dis[:, None] * (acc_ref[0] + acc_ref[1] + hp_ref[...]) + b_ref[...]
    xn = _ln_relu(x_ref[...] + o, g_ref[...], be_ref[...])
    xn_ref[...] = xn
    hpn_ref[...] = jnp.dot(xn, wn_ref[...],
                           preferred_element_type=jnp.float32) * dis[:, None]
    hpnb_ref[...] = (jnp.dot(xn, whn_ref[...],
                             preferred_element_type=jnp.float32)
                     * dis[:, None]).astype(jnp.bfloat16)
    jk = jnp.dot(xn, wjk_ref[...], preferred_element_type=jnp.float32)
    if not first:
        jk = jk + jk_ref[...]
    jko_ref[...] = jk


def _tc_final_body(degp_ref, x_ref, hp_ref, acc_ref, b_ref, g_ref, be_ref,
                   wjk_ref, bjk_ref, jk_ref, out_ref):
    dis = _dis_from(degp_ref[...])
    o = dis[:, None] * (acc_ref[0] + acc_ref[1] + hp_ref[...]) + b_ref[...]
    xn = _ln_relu(x_ref[...] + o, g_ref[...], be_ref[...])
    out_ref[...] = (jk_ref[...] + bjk_ref[...]
                    + jnp.dot(xn, wjk_ref[...],
                              preferred_element_type=jnp.float32))


_b_degp = pl.BlockSpec((BN, NW), lambda i: (i, 0))
_b_rows = pl.BlockSpec((BN, D), lambda i: (i, 0))
_b_acc = pl.BlockSpec((NC, BN, D), lambda i: (0, i, 0))
_b_w = pl.BlockSpec((D, D), lambda i: (0, 0))
_b_vec = pl.BlockSpec((1, D), lambda i: (0, 0))

_f32 = jnp.float32
_nd = jax.ShapeDtypeStruct((N, D), _f32)
_ndb = jax.ShapeDtypeStruct((N, D), jnp.bfloat16)

_tc_prep = pl.pallas_call(
    _tc_prep_body,
    grid=(GRID,),
    in_specs=[_b_degp, _b_rows, _b_w, _b_w],
    out_specs=[_b_rows, _b_rows],
    out_shape=[_nd, _ndb],
)


def _make_post(first):
    return pl.pallas_call(
        functools.partial(_tc_post_body, first=first),
        grid=(GRID,),
        in_specs=[_b_degp, _b_rows, _b_rows, _b_acc, _b_vec, _b_vec, _b_vec,
                  _b_w, _b_w, _b_w, _b_rows],
        out_specs=[_b_rows, _b_rows, _b_rows, _b_rows],
        out_shape=[_nd, _nd, _ndb, _nd],
    )


_tc_post0 = _make_post(True)
_tc_post1 = _make_post(False)

_tc_final = pl.pallas_call(
    _tc_final_body,
    grid=(GRID,),
    in_specs=[_b_degp, _b_rows, _b_rows, _b_acc, _b_vec, _b_vec, _b_vec,
              _b_w, _b_vec, _b_rows],
    out_specs=_b_rows,
    out_shape=_nd,
)


def kernel(node, edge_index, edge_attr, batch_ptr,
           W0, b0, g0, be0, W1, b1, g1, be1, W2, b2, g2, be2,
           Wjk, bjk):
    del batch_ptr
    row = edge_index[0].astype(jnp.int32)
    col = edge_index[1].astype(jnp.int32)
    w = edge_attr.astype(jnp.float32)

    degp = _deg_kernel(col, w).T

    pad = EPAD - E
    rowp = jnp.concatenate([row, jnp.zeros((pad,), jnp.int32)])
    colp = jnp.concatenate([col, jnp.zeros((pad,), jnp.int32)])
    wp = jnp.concatenate([w, jnp.zeros((pad,), jnp.float32)])
    rc = jnp.stack([rowp.reshape(NW * CPB, CH),
                    colp.reshape(NW * CPB, CH)], axis=1)
    wchunk = wp.reshape(NW * CPB, CH)

    b0r, g0r, be0r = b0.reshape(1, D), g0.reshape(1, D), be0.reshape(1, D)
    b1r, g1r, be1r = b1.reshape(1, D), g1.reshape(1, D), be1.reshape(1, D)
    b2r, g2r, be2r = b2.reshape(1, D), g2.reshape(1, D), be2.reshape(1, D)
    wjk0, wjk1, wjk2 = Wjk[:D], Wjk[D:2 * D], Wjk[2 * D:]
    bjkr = bjk.reshape(1, D)
    hperm = jnp.asarray(_H)
    wh0, wh1, wh2 = W0[:, hperm], W1[:, hperm], W2[:, hperm]

    hp0, hpb0 = _tc_prep(degp, node, W0, wh0)
    acc0 = _agg_kernel(hpb0, rc, wchunk)
    x1, hp1, hpb1, jk = _tc_post0(degp, node, hp0, acc0, b0r, g0r, be0r,
                                  W1, wh1, wjk0, jnp.zeros((N, D), _f32))
    acc1 = _agg_kernel(hpb1, rc, wchunk)
    x2, hp2, hpb2, jk = _tc_post1(degp, x1, hp1, acc1, b1r, g1r, be1r,
                                  W2, wh2, wjk1, jk)
    acc2 = _agg_kernel(hpb2, rc, wchunk)
    return _tc_final(degp, x2, hp2, acc2, b2r, g2r, be2r, wjk2, bjkr, jk)
